# Initial kernel scaffold; baseline (speedup 1.0000x reference)
import jax
import jax.numpy as jnp
from jax.experimental import pallas as pl


def kernel(input_, image):
    raise NotImplementedError("write your pallas kernel here")


# speedup vs baseline: 2.0941x; 2.0941x over previous
import numpy as np
import jax
import jax.numpy as jnp
from jax.experimental import pallas as pl

_EPS = float(np.finfo('float').eps)
_INV_GAMMA = 1.0 / 5.0
_N = 4096
_W = 64
_TILE = 256


def _coords(n):
    x = jax.lax.rem(n, _W).astype(jnp.float32) * _INV_GAMMA
    y = jax.lax.div(n, _W).astype(jnp.float32) * _INV_GAMMA
    return x, y


def _dense_kernel(q2_ref, o_ref):
    t = pl.program_id(0)
    row = jax.lax.broadcasted_iota(jnp.int32, (_TILE, 1), 0) + t * _TILE
    col = jax.lax.broadcasted_iota(jnp.int32, (1, _N), 1)
    xi, yi = _coords(row)
    xj, yj = _coords(col)
    bxi = xi.astype(jnp.bfloat16).astype(jnp.float32)
    byi = yi.astype(jnp.bfloat16).astype(jnp.float32)
    bxj = xj.astype(jnp.bfloat16).astype(jnp.float32)
    byj = yj.astype(jnp.bfloat16).astype(jnp.float32)
    sqi = xi * xi + yi * yi
    sqj = xj * xj + yj * yj
    dot = bxi * bxj + byi * byj
    d2 = jnp.maximum((sqi + sqj) - 2.0 * dot, 0.0)
    w_tile = jnp.exp(-0.5 * d2)
    o = jnp.dot(w_tile, q2_ref[...], preferred_element_type=jnp.float32)
    o_ref[...] = o[:, :21] * (1.0 / (o[:, 21:22] + _EPS))


def kernel(input_, image):
    _, d, h, w = input_.shape
    flat = input_.reshape(d, h * w).T
    q2 = jnp.concatenate([flat, jnp.ones((h * w, 1), jnp.float32)], axis=1)
    out = pl.pallas_call(
        _dense_kernel,
        grid=(_N // _TILE,),
        in_specs=[pl.BlockSpec((_N, d + 1), lambda t: (0, 0))],
        out_specs=pl.BlockSpec((_TILE, d), lambda t: (t, 0)),
        out_shape=jax.ShapeDtypeStruct((_N, d), jnp.float32),
    )(q2)
    return out.T.reshape(1, d, h, w)


# speedup vs baseline: 2.9016x; 1.3856x over previous
import numpy as np
import ml_dtypes
import jax
import jax.numpy as jnp
from jax.experimental import pallas as pl

_EPS = float(np.finfo('float').eps)
_HW = 64
_R = 6


def _build_tables():
    e = (np.arange(_HW, dtype=np.float32) / np.float32(5.0)).astype(np.float32)
    b = e.astype(ml_dtypes.bfloat16).astype(np.float32)
    d2 = (e[:, None] * e[:, None] + e[None, :] * e[None, :]
          - np.float32(2.0) * (b[:, None] * b[None, :])).astype(np.float32)
    gb = np.exp(-0.5 * d2.astype(np.float64)).astype(np.float32)

    offsets = []
    corrs = []
    idx = np.arange(_HW)
    for dy in range(-_R, _R + 1):
        jy = idx + dy
        vy = (jy >= 0) & (jy < _HW)
        u = np.full(_HW, np.inf, np.float32)
        u[vy] = d2[idx[vy], jy[vy]]
        for dx in range(-_R, _R + 1):
            jx = idx + dx
            vx = (jx >= 0) & (jx < _HW)
            v = np.full(_HW, np.inf, np.float32)
            v[vx] = d2[idx[vx], jx[vx]]
            s = u[:, None] + v[None, :]
            neg = s < 0
            if neg.any():
                c = np.zeros((_HW, _HW), np.float64)
                c[neg] = 1.0 - np.exp(-0.5 * s[neg].astype(np.float64))
                offsets.append((dy, dx))
                corrs.append(c.astype(np.float32))
    order = sorted(range(len(offsets)), key=lambda i: (offsets[i][1], offsets[i][0]))
    offsets = [offsets[i] for i in order]
    corrs = [corrs[i] for i in order]
    corr = np.stack(corrs)

    sg = gb.astype(np.float64).sum(axis=1)
    norm = np.outer(sg, sg) + corr.astype(np.float64).sum(axis=0)
    inv_norm = (1.0 / (norm + _EPS)).astype(np.float32)
    return gb, offsets, corr, inv_norm


_GB_NP, _OFFSETS, _CORR_NP, _INVN_NP = _build_tables()


def _filter_kernel(qpad_ref, gb_ref, corr_ref, invn_ref, o_ref):
    d = o_ref.shape[0]
    qpad = qpad_ref[...]
    gb = gb_ref[...]
    q = qpad[:, _R:_R + _HW, _R:_R + _HW]

    t1 = jnp.dot(q.reshape(d * _HW, _HW), gb,
                 preferred_element_type=jnp.float32).reshape(d, _HW, _HW)
    acc = jnp.stack([
        jnp.dot(gb, t1[c], preferred_element_type=jnp.float32)
        for c in range(d)
    ])

    k = 0
    last_dx = None
    qx = None
    for dy, dx in _OFFSETS:
        if dx != last_dx:
            qx = qpad[:, :, _R + dx:_R + dx + _HW]
            last_dx = dx
        qs = qx[:, _R + dy:_R + dy + _HW, :]
        acc = acc + corr_ref[k][None, :, :] * qs
        k += 1

    o_ref[...] = acc * invn_ref[...][None, :, :]


def kernel(input_, image):
    _, d, h, w = input_.shape
    q = input_.reshape(d, h, w)
    qpad = jnp.pad(q, ((0, 0), (_R, _R), (_R, _R)))
    out = pl.pallas_call(
        _filter_kernel,
        out_shape=jax.ShapeDtypeStruct((d, h, w), jnp.float32),
    )(qpad, jnp.asarray(_GB_NP), jnp.asarray(_CORR_NP), jnp.asarray(_INVN_NP))
    return out.reshape(1, d, h, w)
